# Initial kernel scaffold; baseline (speedup 1.0000x reference)
import jax
import jax.numpy as jnp
from jax.experimental import pallas as pl


def kernel(x, w_t, gamma, beta):
    raise NotImplementedError("write your pallas kernel here")


# speedup vs baseline: 6.0344x; 6.0344x over previous
import functools

import jax
import jax.numpy as jnp
from jax.experimental import pallas as pl
from jax.experimental.pallas import tpu as pltpu

BN_EPS = 1e-5
LANE = 128
SUBLANE = 8
VMEM_LIMIT = 60 * 1024 * 1024


def _fused_kernel(x_ref, w_ref, gamma_ref, beta_ref, o_ref,
                  xc_ref, acc_ref, ab_ref, *,
                  bn, n_chunks, pad, ho, wo, inv_cnt):
    p = pl.program_id(0)
    b = pl.program_id(1)

    @pl.when(p == 0)
    def _phase0():
        @pl.when(b == 0)
        def _():
            acc_ref[...] = jnp.zeros_like(acc_ref)

        xc = x_ref[:, pad:pad + ho, pad:pad + wo]
        xc_ref[pl.ds(b * bn, bn)] = xc
        v = xc.reshape(-1, SUBLANE, LANE)
        acc_ref[0] += jnp.sum(v, axis=0)
        acc_ref[1] += jnp.sum(v * v, axis=0)

        @pl.when(b == n_chunks - 1)
        def _finalize():
            s1 = jnp.sum(acc_ref[0])
            s2 = jnp.sum(acc_ref[1])
            w = w_ref[0]
            mean_t = w * s1 * inv_cnt
            ex2_t = w * w * s2 * inv_cnt
            var = jnp.maximum(ex2_t - mean_t * mean_t, 0.0)
            a = gamma_ref[0] * jax.lax.rsqrt(var + BN_EPS)
            ab_ref[0] = w * a
            ab_ref[1] = beta_ref[0] - mean_t * a

    @pl.when(p == 1)
    def _phase1():
        a = ab_ref[0]
        c = ab_ref[1]
        xc = xc_ref[pl.ds(b * bn, bn)]
        o_ref[...] = jnp.maximum(xc * a + c, 0.0)


@functools.partial(jax.jit, static_argnames=("stride", "padding"))
def _forward(x, w_t, gamma, beta, *, stride=1, padding=1):
    N, Cin, H, W = x.shape
    Cin_w, Cout, kH, kW = w_t.shape
    assert Cin == 1 and Cout == 1 and kH == 1 and kW == 1 and stride == 1

    Ho = (H - 1) * stride - 2 * padding + kH
    Wo = (W - 1) * stride - 2 * padding + kW
    assert Ho > 0 and Wo > 0

    bn = 1
    for cand in (32, 16, 8, 4, 2):
        if N % cand == 0 and (cand * Ho * Wo) % (SUBLANE * LANE) == 0:
            bn = cand
            break
    assert (bn * Ho * Wo) % (SUBLANE * LANE) == 0
    n_chunks = N // bn

    x3 = x.reshape(N, H, W)
    w1 = w_t.reshape(1).astype(jnp.float32)
    gamma32 = gamma.astype(jnp.float32)
    beta32 = beta.astype(jnp.float32)
    inv_cnt = 1.0 / float(N * Ho * Wo)

    out = pl.pallas_call(
        functools.partial(_fused_kernel, bn=bn, n_chunks=n_chunks,
                          pad=padding, ho=Ho, wo=Wo, inv_cnt=inv_cnt),
        out_shape=jax.ShapeDtypeStruct((N, Ho, Wo), x.dtype),
        grid=(2, n_chunks),
        in_specs=[
            pl.BlockSpec((bn, H, W),
                         lambda p, b: (b * (1 - p) + (n_chunks - 1) * p, 0, 0)),
            pl.BlockSpec(memory_space=pltpu.MemorySpace.SMEM),
            pl.BlockSpec(memory_space=pltpu.MemorySpace.SMEM),
            pl.BlockSpec(memory_space=pltpu.MemorySpace.SMEM),
        ],
        out_specs=pl.BlockSpec((bn, Ho, Wo), lambda p, b: (b * p, 0, 0)),
        scratch_shapes=[
            pltpu.VMEM((N, Ho, Wo), jnp.float32),
            pltpu.VMEM((2, SUBLANE, LANE), jnp.float32),
            pltpu.SMEM((2,), jnp.float32),
        ],
        compiler_params=pltpu.CompilerParams(
            dimension_semantics=("arbitrary", "arbitrary"),
            vmem_limit_bytes=VMEM_LIMIT),
    )(x3, w1, gamma32, beta32)

    return out.reshape(N, Cout, Ho, Wo)


def kernel(x, w_t, gamma, beta):
    return _forward(x, w_t, gamma, beta, stride=1, padding=1)
